# baseline (speedup 1.0000x reference)
import functools

import jax
import jax.numpy as jnp
from jax import lax
from jax.experimental import pallas as pl
from jax.experimental.pallas import tpu as pltpu
from jax.experimental.pallas import tpu_sc as plsc

B = 16384
D = 64
S = 20
VOCAB = 100000
SUBVOCAB = 1000000
NW = 32
BPW = B // NW
CB = 16
NCHUNK = BPW // CB
SGC = CB * S // 64
SIR = BPW * S // 64
WIR = BPW // 16


@functools.partial(
    pl.kernel,
    mesh=plsc.VectorSubcoreMesh(core_axis_name="c", subcore_axis_name="s"),
    compiler_params=pltpu.CompilerParams(use_tc_tiling_on_sc=False),
    out_type=jax.ShapeDtypeStruct((B, D), jnp.float32),
    scratch_types=[
        pltpu.VMEM((WIR, 16), jnp.int32),
        pltpu.VMEM((WIR * 16 + 16,), jnp.int32),
        pltpu.VMEM((SIR, 64), jnp.int32),
        pltpu.VMEM((SIR * 64 + 32,), jnp.int32),
        pltpu.VMEM((CB * S, 2 * D), jnp.float32),
        pltpu.VMEM((CB * S, 2 * D), jnp.float32),
        pltpu.VMEM((CB, 2 * D), jnp.float32),
        pltpu.VMEM((CB, 2 * D), jnp.float32),
        pltpu.VMEM((CB, D), jnp.float32),
        pltpu.VMEM((CB, D), jnp.float32),
        pltpu.SemaphoreType.DMA,
        pltpu.SemaphoreType.DMA,
        pltpu.SemaphoreType.DMA,
        pltpu.SemaphoreType.DMA,
        pltpu.SemaphoreType.DMA,
        pltpu.SemaphoreType.DMA,
    ],
)
def _fasttext_sc(widx_hbm, sidx_hbm, wemb_hbm, semb_hbm, out_hbm,
                 wj, wp, sj, sp, gbuf0, gbuf1, wbuf0, wbuf1, outb0, outb1,
                 sem0, sem1, wsem0, wsem1, osem0, osem1):
    wid = lax.axis_index("s") * 2 + lax.axis_index("c")
    base = wid * BPW

    pltpu.sync_copy(widx_hbm.at[pl.ds(wid * WIR, WIR), :], wj)
    pltpu.sync_copy(sidx_hbm.at[pl.ds(wid * SIR, SIR), :], sj)

    one = jnp.int32(1)

    def sprep(i, carry):
        for q in range(4):
            x = sj[i, pl.ds(q * 16, 16)]
            sj[i, pl.ds(q * 16, 16)] = lax.shift_right_logical(x, one)
            sp[pl.ds(i * 64 + q * 16, 16)] = lax.shift_left(
                lax.bitwise_and(x, one), jnp.int32(6))
        return carry

    lax.fori_loop(0, SIR, sprep, 0)

    def wprep(i, carry):
        x = wj[i, :]
        wj[i, :] = lax.shift_right_logical(x, one)
        wp[pl.ds(i * 16, 16)] = lax.shift_left(
            lax.bitwise_and(x, one), jnp.int32(6))
        return carry

    lax.fori_loop(0, WIR, wprep, 0)

    gbufs = [gbuf0, gbuf1]
    wbufs = [wbuf0, wbuf1]
    outbs = [outb0, outb1]
    sems = [sem0, sem1]
    wsems = [wsem0, wsem1]
    osems = [osem0, osem1]

    for half in range(2):
        for i in range(SGC):
            pltpu.async_copy(semb_hbm.at[sj.at[half * SGC + i]],
                             gbufs[half].at[pl.ds(i * 64, 64), :],
                             sems[half])
        pltpu.async_copy(wemb_hbm.at[wj.at[half]], wbufs[half], wsems[half])

    def outer(p, carry):
        for half in range(2):
            c = p * 2 + half
            gb = gbufs[half]
            wb = wbufs[half]
            ob = outbs[half]
            for i in range(SGC):
                pltpu.make_async_copy(semb_hbm.at[pl.ds(0, 64), :],
                                      gb.at[pl.ds(i * 64, 64), :],
                                      sems[half]).wait()
            pltpu.make_async_copy(wemb_hbm.at[pl.ds(0, CB), :], wb,
                                  wsems[half]).wait()

            @pl.when(c >= 2)
            def _(ob=ob, half=half):
                pltpu.make_async_copy(
                    ob, out_hbm.at[pl.ds(base, CB), :], osems[half]).wait()

            def body(b, inner_carry, c=c, gb=gb, wb=wb, ob=ob):
                fbase = c * (CB * S) + b * S
                v0 = sp[pl.ds(fbase, 16)]
                v1 = sp[pl.ds(fbase + 16, 16)]
                po = [v0[s] for s in range(16)] + [v1[s] for s in range(S - 16)]
                wpo = wp[pl.ds(c * CB + b, 16)][0]
                for k in range(D // 16):
                    acc = wb[b, pl.ds(wpo + k * 16, 16)] * jnp.float32(S)
                    for s in range(S):
                        acc = acc + gb[b * S + s, pl.ds(po[s] + k * 16, 16)]
                    ob[b, pl.ds(k * 16, 16)] = acc * jnp.float32(0.5 / S)
                return inner_carry

            lax.fori_loop(0, CB, body, 0)

            pltpu.async_copy(ob, out_hbm.at[pl.ds(base + c * CB, CB), :],
                             osems[half])

            @pl.when(c + 2 < NCHUNK)
            def _(c=c, gb=gb, wb=wb, half=half):
                for i in range(SGC):
                    pltpu.async_copy(semb_hbm.at[sj.at[(c + 2) * SGC + i]],
                                     gb.at[pl.ds(i * 64, 64), :], sems[half])
                pltpu.async_copy(wemb_hbm.at[wj.at[c + 2]], wb, wsems[half])
        return carry

    lax.fori_loop(0, NCHUNK // 2, outer, 0)

    for half in range(2):
        pltpu.make_async_copy(outbs[half], out_hbm.at[pl.ds(base, CB), :],
                              osems[half]).wait()


def kernel(word_idx, subword_idx, word_emb, subword_emb):
    widx = word_idx.astype(jnp.int32).reshape(B // 16, 16)
    sidx = subword_idx.astype(jnp.int32).reshape(B * S // 64, 64)
    wemb2 = word_emb.reshape(VOCAB // 2, 2 * D)
    semb2 = subword_emb.reshape(SUBVOCAB // 2, 2 * D)
    return _fasttext_sc(widx, sidx, wemb2, semb2)
